# Initial kernel scaffold; baseline (speedup 1.0000x reference)
import jax
import jax.numpy as jnp
from jax.experimental import pallas as pl


def kernel(x, edge_index, batch, W0, b0, W1, b1, W2, b2, W3, b3, W4, b4, Wc1, bc1, Wc2, bc2, Wd1, bd1, Wd2, bd2):
    raise NotImplementedError("write your pallas kernel here")


# speedup vs baseline: 5.7008x; 5.7008x over previous
import functools

import jax
import jax.numpy as jnp
from jax import lax
from jax.experimental import pallas as pl
from jax.experimental.pallas import tpu as pltpu
from jax.experimental.pallas import tpu_sc as plsc

N = 10000
NR = 10016
NRK = 10112
E = 320000
NC = 2
NS = 16
NW = NC * NS
CS = 128
CW = 80
EPAD = NW * CW * CS
D = 32
DUMP = 10008
ZROW = 10008
NGRAPH = 64
K = 79
PG = 80
NPOOL = NGRAPH * PG
DC = 144
TPN = NR // NS

_f32 = jnp.float32
_i32 = jnp.int32


@functools.cache
def _mesh():
    return plsc.VectorSubcoreMesh(core_axis_name="c", subcore_axis_name="s")


_SC_PARAMS = None


def _sc_params():
    global _SC_PARAMS
    if _SC_PARAMS is None:
        _SC_PARAMS = pltpu.CompilerParams(use_tc_tiling_on_sc=False)
    return _SC_PARAMS


def _worker_id():
    return lax.axis_index("s") * NC + lax.axis_index("c")


def _zero_rows(buf, rows, width):
    z16 = jnp.zeros((16,), _f32)

    def body(r, _):
        for c0 in range(0, width, 16):
            buf[r, pl.ds(c0, 16)] = z16
        return 0

    lax.fori_loop(0, rows, body, 0)


def _make_scatter_kernel(with_deg):
  out_type = jax.ShapeDtypeStruct((NC, NR, D), _f32)
  scratch = [
      pltpu.VMEM((CW, CS), _i32),
      pltpu.VMEM((CW, CS), _i32),
      pltpu.VMEM((CS, D), _f32),
      pltpu.VMEM((TPN, D), _f32),
      pltpu.VMEM_SHARED((NR, D), _f32),
      pltpu.SemaphoreType.DMA,
  ]
  if with_deg:
    out_type = (out_type, jax.ShapeDtypeStruct((NC, NR, 16), _f32))
    scratch += [
        pltpu.VMEM((CS, 16), _f32),
        pltpu.VMEM((TPN, 16), _f32),
        pltpu.VMEM_SHARED((NR, 16), _f32),
    ]

  @functools.partial(
      pl.kernel,
      out_type=out_type,
      mesh=_mesh(),
      scratch_types=scratch,
      compiler_params=_sc_params(),
  )
  def _sc_scatter(table, gidx, sidx, out, *rest):
    if with_deg:
      (odeg, gidx_v, sidx_v, rows_v, obuf_v, accum_sh, sem, ones_v, dbuf_v,
       dacc_sh) = rest
    else:
      gidx_v, sidx_v, rows_v, obuf_v, accum_sh, sem = rest
    c = lax.axis_index("c")
    s = lax.axis_index("s")
    wid = _worker_id()
    _zero_rows(obuf_v, TPN, D)
    pltpu.sync_copy(obuf_v, accum_sh.at[pl.ds(s * TPN, TPN)])
    if with_deg:
      one16 = jnp.ones((16,), _f32)

      def fill(r, _):
          ones_v[r, pl.ds(0, 16)] = one16
          return 0

      lax.fori_loop(0, CS, fill, 0)
      _zero_rows(dbuf_v, TPN, 16)
      pltpu.sync_copy(dbuf_v, dacc_sh.at[pl.ds(s * TPN, TPN)])
    pltpu.sync_copy(gidx.at[pl.ds(wid * CW, CW)], gidx_v)
    pltpu.sync_copy(sidx.at[pl.ds(wid * CW, CW)], sidx_v)
    plsc.subcore_barrier()

    def chunk(j, _):
        pltpu.async_copy(table.at[gidx_v.at[j]], rows_v, sem).wait()
        pltpu.sync_copy(rows_v, accum_sh.at[sidx_v.at[j]], add=True)
        if with_deg:
            pltpu.sync_copy(ones_v, dacc_sh.at[gidx_v.at[j]], add=True)
        return 0

    lax.fori_loop(0, CW, chunk, 0)
    plsc.subcore_barrier()
    pltpu.sync_copy(accum_sh.at[pl.ds(s * TPN, TPN)], obuf_v)
    pltpu.sync_copy(obuf_v, out.at[c].at[pl.ds(s * TPN, TPN)])
    if with_deg:
      pltpu.sync_copy(dacc_sh.at[pl.ds(s * TPN, TPN)], dbuf_v)
      pltpu.sync_copy(dbuf_v, odeg.at[c].at[pl.ds(s * TPN, TPN)])

  return _sc_scatter


@functools.cache
def _sc_scatter_kernel():
  return _make_scatter_kernel(False)


@functools.cache
def _sc_scatter_deg_kernel():
  return _make_scatter_kernel(True)


@functools.cache
def _sc_poolgather_kernel():
  @functools.partial(
      pl.kernel,
      out_type=jax.ShapeDtypeStruct((NPOOL, DC), _f32),
      mesh=_mesh(),
      scratch_types=[
          pltpu.VMEM((1, 80), _i32),
          pltpu.VMEM((80, DC), _f32),
          pltpu.SemaphoreType.DMA,
      ],
      compiler_params=_sc_params(),
  )
  def _sc_poolgather(cat, nid2d, out, nid_v, rows_v, sem):
    wid = _worker_id()

    def body(k, _):
        row = wid * 2 + k
        pltpu.sync_copy(nid2d.at[pl.ds(row, 1)], nid_v)
        pltpu.async_copy(cat.at[nid_v.at[0]], rows_v, sem).wait()
        pltpu.sync_copy(rows_v, out.at[pl.ds(row * 80, 80)])
        return 0

    lax.fori_loop(0, 2, body, 0)

  return _sc_poolgather


def _tc0_body(x_ref, w_ref, b_ref, h_ref):
    h_ref[...] = (
        jnp.dot(x_ref[...], w_ref[...], preferred_element_type=_f32)
        + b_ref[...]
    )


def _tc_mid1_body(s_ref, od_ref, h_ref, w_ref, b_ref, invd_ref, t_ref, hn_ref):
    invd = 1.0 / (od_ref[0, :, 0:1] + od_ref[1, :, 0:1] + 1.0)
    invd_ref[...] = invd
    t = jnp.tanh((s_ref[0] + s_ref[1] + h_ref[...]) * invd)
    t_ref[...] = t
    hn_ref[...] = (
        jnp.dot(t, w_ref[...], preferred_element_type=_f32) + b_ref[...]
    )


def _tc_mid_body(s_ref, h_ref, invd_ref, w_ref, b_ref, t_ref, hn_ref):
    t = jnp.tanh((s_ref[0] + s_ref[1] + h_ref[...]) * invd_ref[...])
    t_ref[...] = t
    hn_ref[...] = (
        jnp.dot(t, w_ref[...], preferred_element_type=_f32) + b_ref[...]
    )


def _tc_mid3_body(s_ref, h_ref, invd_ref, w4_ref, b4_ref, t_ref, h4p_ref):
    t = jnp.tanh((s_ref[0] + s_ref[1] + h_ref[...]) * invd_ref[...])
    t_ref[...] = t
    h4 = jnp.dot(t, w4_ref[...], preferred_element_type=_f32) + b4_ref[...]
    h4p_ref[...] = jnp.concatenate([h4, jnp.zeros((NR, D - 1), _f32)], axis=1)


TC4R = NR // 4


def _tc4_body(s_ref, h4p_ref, invd_ref, t0_ref, t1_ref, t2_ref, t3_ref,
              cat_ref):
    i = pl.program_id(0)
    pre4 = (s_ref[0, :, 0:1] + s_ref[1, :, 0:1] + h4p_ref[:, 0:1]) * invd_ref[...]
    t4 = jnp.tanh(pre4)
    cat = jnp.concatenate(
        [t0_ref[...], t1_ref[...], t2_ref[...], t3_ref[...], t4,
         jnp.zeros((TC4R, DC - 129), _f32)], axis=1)
    rmask = (lax.broadcasted_iota(_i32, (TC4R, DC), 0) + i * TC4R) < N
    cat_ref[...] = jnp.where(rmask, cat, 0.0)


def _rank_body(b2_ref, v2_ref, bc_ref, vc_ref, pos_ref):
    i = pl.program_id(0)
    bi = b2_ref[pl.ds(i, 1), :]
    vi = v2_ref[pl.ds(i, 1), :]
    gi_lo = jnp.min(bi)
    gi_hi = jnp.max(bi)
    col = lax.broadcasted_iota(_i32, (128, 128), 1) + i * 128

    def jbody(j, acc):
        bj_row = b2_ref[pl.ds(j, 1), :]
        gj_lo = jnp.min(bj_row)
        gj_hi = jnp.max(bj_row)
        ok = (gj_lo <= gi_hi) & (gj_hi >= gi_lo)

        def compute():
            vj = vc_ref[pl.ds(j * 128, 128), :]
            bj = bc_ref[pl.ds(j * 128, 128), :]
            row = lax.broadcasted_iota(_i32, (128, 128), 0) + j * 128
            m = (bj == bi) & ((vj > vi) | ((vj == vi) & (row < col)))
            return acc + jnp.sum(m.astype(_f32), axis=0, keepdims=True)

        return lax.cond(ok, compute, lambda: acc)

    rank = lax.fori_loop(0, NRK // 128, jbody,
                         jnp.zeros((1, 128), _f32)).astype(_i32)
    valid = (bi < NGRAPH) & (rank < K)
    pos = jnp.where(valid, bi * PG + rank, NPOOL)
    pos_ref[...] = pos.reshape(1, 1, 128)


def _nid_body(pc_ref, b2_ref, o_ref):
    b = pl.program_id(0)
    slots = lax.broadcasted_iota(_i32, (1, 512), 1) + b * 512

    def jbody(j, acc):
        bj = b2_ref[pl.ds(j, 1), :]
        lo = jnp.min(bj) * PG
        hi = jnp.max(bj) * PG + (PG - 1)
        ok = (lo <= b * 512 + 511) & (hi >= b * 512)

        def compute():
            pcol = pc_ref[pl.ds(j * 128, 128), :]
            ids = lax.broadcasted_iota(_i32, (128, 1), 0) + j * 128
            m = pcol == slots
            contrib = jnp.where(m, ids, -1)
            return jnp.maximum(acc, jnp.max(contrib, axis=0, keepdims=True))

        return lax.cond(ok, compute, lambda: acc)

    acc = lax.fori_loop(0, NRK // 128, jbody, jnp.full((1, 512), -1, _i32))
    o_ref[...] = jnp.where(acc < 0, ZROW, acc).reshape(1, 1, 512)


def _tail_body(p_ref, wc1_ref, bc1_ref, w2_ref, bc2_ref, wd1_ref, bd1_ref,
               wd2_ref, bd2_ref, o_ref):
    p = p_ref[...].reshape(NGRAPH, PG, DC)[:, :K, :129]
    c1 = jax.nn.relu(
        jnp.dot(p.reshape(NGRAPH * K, 129), wc1_ref[...],
                preferred_element_type=_f32) + bc1_ref[...]
    ).reshape(NGRAPH, K, 16)
    cp = jnp.max(c1[:, :78, :].reshape(NGRAPH, 39, 2, 16), axis=2)
    xw = jnp.concatenate([cp[:, dt:dt + 35, :] for dt in range(5)], axis=2)
    c2 = jax.nn.relu(
        jnp.dot(xw.reshape(NGRAPH * 35, 80), w2_ref[...],
                preferred_element_type=_f32) + bc2_ref[...]
    ).reshape(NGRAPH, 35, 32)
    acc = bd1_ref[...]
    for t in range(35):
        acc = acc + jnp.dot(c2[:, t, :], wd1_ref[pl.ds(t * 32, 32), :],
                            preferred_element_type=_f32)
    d1 = jax.nn.relu(acc)
    o_ref[...] = jax.nn.sigmoid(
        jnp.dot(d1, wd2_ref[...], preferred_element_type=_f32)
        + bd2_ref[...])


def _call(body, out_shape, *args):
    return pl.pallas_call(body, out_shape=out_shape)(*args)


def kernel(x, edge_index, batch, W0, b0, W1, b1, W2, b2, W3, b3, W4, b4,
           Wc1, bc1, Wc2, bc2, Wd1, bd1, Wd2, bd2):
    sds = jax.ShapeDtypeStruct

    xp = jnp.pad(x, ((0, NR - N), (0, 0)))
    src = edge_index[0]
    dst = edge_index[1]
    g2d = jnp.concatenate(
        [src, jnp.full((EPAD - E,), DUMP, _i32)]).reshape(NW * CW, CS)
    s2d = jnp.concatenate(
        [dst, jnp.full((EPAD - E,), DUMP, _i32)]).reshape(NW * CW, CS)
    batchp = jnp.pad(batch, (0, NRK - N), constant_values=127)

    h0 = _call(_tc0_body, sds((NR, D), _f32), xp, W0.T, b0.reshape(1, D))
    s0, odp = _sc_scatter_deg_kernel()(h0, g2d, s2d)
    invd, t0, h1 = _call(
        _tc_mid1_body,
        (sds((NR, 1), _f32), sds((NR, D), _f32), sds((NR, D), _f32)),
        s0, odp, h0, W1.T, b1.reshape(1, D))
    s1 = _sc_scatter_kernel()(h1, g2d, s2d)
    t1, h2 = _call(
        _tc_mid_body, (sds((NR, D), _f32), sds((NR, D), _f32)),
        s1, h1, invd, W2.T, b2.reshape(1, D))
    s2 = _sc_scatter_kernel()(h2, g2d, s2d)
    t2, h3 = _call(
        _tc_mid_body, (sds((NR, D), _f32), sds((NR, D), _f32)),
        s2, h2, invd, W3.T, b3.reshape(1, D))
    s3 = _sc_scatter_kernel()(h3, g2d, s2d)
    t3, h4p = _call(
        _tc_mid3_body, (sds((NR, D), _f32), sds((NR, D), _f32)),
        s3, h3, invd, W4.T, b4.reshape(1, 1))
    s4 = _sc_scatter_kernel()(h4p, g2d, s2d)
    cat = pl.pallas_call(
        _tc4_body,
        out_shape=sds((NR, DC), _f32),
        grid=(NR // TC4R,),
        in_specs=[
            pl.BlockSpec((NC, TC4R, D), lambda i: (0, i, 0)),
            pl.BlockSpec((TC4R, D), lambda i: (i, 0)),
            pl.BlockSpec((TC4R, 1), lambda i: (i, 0)),
            pl.BlockSpec((TC4R, D), lambda i: (i, 0)),
            pl.BlockSpec((TC4R, D), lambda i: (i, 0)),
            pl.BlockSpec((TC4R, D), lambda i: (i, 0)),
            pl.BlockSpec((TC4R, D), lambda i: (i, 0)),
        ],
        out_specs=pl.BlockSpec((TC4R, DC), lambda i: (i, 0)),
    )(s4, h4p, invd, t0, t1, t2, t3)

    b2d = batchp.reshape(NRK // 128, 128)
    vpad = jnp.pad(cat[:, 128], (0, NRK - NR))
    v2d = vpad.reshape(NRK // 128, 128)
    pos3 = pl.pallas_call(
        _rank_body,
        out_shape=sds((NRK // 128, 1, 128), _i32),
        grid=(NRK // 128,),
        in_specs=[
            pl.BlockSpec((NRK // 128, 128), lambda i: (0, 0)),
            pl.BlockSpec((NRK // 128, 128), lambda i: (0, 0)),
            pl.BlockSpec((NRK, 1), lambda i: (0, 0)),
            pl.BlockSpec((NRK, 1), lambda i: (0, 0)),
        ],
        out_specs=pl.BlockSpec((1, 1, 128), lambda i: (i, 0, 0)),
    )(b2d, v2d, batchp.reshape(NRK, 1), vpad.reshape(NRK, 1))

    posc = pos3.reshape(NRK, 1)
    nid3 = pl.pallas_call(
        _nid_body,
        out_shape=sds((NPOOL // 512, 1, 512), _i32),
        grid=(NPOOL // 512,),
        in_specs=[
            pl.BlockSpec((NRK, 1), lambda i: (0, 0)),
            pl.BlockSpec((NRK // 128, 128), lambda i: (0, 0)),
        ],
        out_specs=pl.BlockSpec((1, 1, 512), lambda i: (i, 0, 0)),
    )(posc, b2d)
    nid2d = nid3.reshape(NGRAPH, PG)
    pooled = _sc_poolgather_kernel()(cat, nid2d)

    wc1f = Wc1[:, 0, :].T
    w2f = Wc2.transpose(2, 1, 0).reshape(80, 32)
    wd1p = Wd1.reshape(128, 32, 35).transpose(2, 1, 0).reshape(1120, 128)
    out = _call(
        _tail_body, sds((NGRAPH, 10), _f32),
        pooled, wc1f, bc1.reshape(1, 16), w2f, bc2.reshape(1, 32),
        wd1p, bd1.reshape(1, 128), Wd2.T, bd2.reshape(1, 10))
    return out
